# Initial kernel scaffold; baseline (speedup 1.0000x reference)
import jax
import jax.numpy as jnp
from jax.experimental import pallas as pl


def kernel(x, router_W, router_b, expert_W, expert_b):
    raise NotImplementedError("write your pallas kernel here")


# speedup vs baseline: 1.6998x; 1.6998x over previous
import jax
import jax.numpy as jnp
from jax.experimental import pallas as pl
from jax.experimental.pallas import tpu as pltpu

_D = 2048
_E = 8
_NB_N = 4


def _router_body(x_ref, rw_ref, rb_ref, coef_ref):
    logits = jnp.dot(x_ref[...], rw_ref[...], preferred_element_type=jnp.float32)
    logits = logits + rb_ref[...]
    m = jnp.max(logits, axis=1, keepdims=True)
    p = jnp.exp(logits - m)
    eio = jax.lax.broadcasted_iota(jnp.int32, p.shape, 1)
    m1 = jnp.max(p, axis=1, keepdims=True)
    i1 = jnp.min(jnp.where(p == m1, eio, _E), axis=1, keepdims=True)
    sel1 = eio == i1
    pm = jnp.where(sel1, -1.0, p)
    m2 = jnp.max(pm, axis=1, keepdims=True)
    i2 = jnp.min(jnp.where(pm == m2, eio, _E), axis=1, keepdims=True)
    sel2 = eio == i2
    s = m1 + m2
    coef_ref[...] = jnp.where(sel1, m1 / s, 0.0) + jnp.where(sel2, m2 / s, 0.0)


def _moe_dense_body(x_ref, w_ref, b_ref, coef_ref, out_ref):
    e = pl.program_id(1)
    eh = (jax.lax.broadcasted_iota(jnp.int32, (_E, 1), 0) == e).astype(jnp.float32)
    c = jnp.dot(coef_ref[...], eh, preferred_element_type=jnp.float32)
    acc = jnp.dot(x_ref[...], w_ref[0], preferred_element_type=jnp.float32)
    acc = acc + b_ref[0]

    @pl.when(e == 0)
    def _init():
        out_ref[...] = jnp.zeros_like(out_ref)

    out_ref[...] += c * acc


def kernel(x, router_W, router_b, expert_W, expert_b):
    B, S, D = x.shape
    T = B * S
    xt = x.reshape(T, D)

    coef = pl.pallas_call(
        _router_body,
        out_shape=jax.ShapeDtypeStruct((T, _E), jnp.float32),
    )(xt, router_W, router_b.reshape(1, _E))

    bn = D // _NB_N
    out = pl.pallas_call(
        _moe_dense_body,
        grid=(_NB_N, _E),
        in_specs=[
            pl.BlockSpec((T, D), lambda n, e: (0, 0)),
            pl.BlockSpec((1, D, bn), lambda n, e: (e, 0, n)),
            pl.BlockSpec((1, 1, bn), lambda n, e: (e, 0, n)),
            pl.BlockSpec((T, _E), lambda n, e: (0, 0)),
        ],
        out_specs=pl.BlockSpec((T, bn), lambda n, e: (0, n)),
        out_shape=jax.ShapeDtypeStruct((T, D), jnp.float32),
    )(xt, expert_W, expert_b.reshape(_E, 1, D), coef)

    return out.reshape(B, S, D)
